# baseline (speedup 1.0000x reference)
import functools

import jax
import jax.numpy as jnp
from jax import lax
from jax.experimental import pallas as pl
from jax.experimental.pallas import tpu as pltpu
from jax.experimental.pallas import tpu_sc as plsc

_N_REF, _N_SRC, _R, _S, _D, _C = 4096, 4096, 512, 512, 128, 2048
_NC, _NS = 2, 16
_NW = _NC * _NS
_RPW = _R // _NW
_RB = 16


def _feat_body(ridx_f_ref, sidx_f_ref, ref_feats_ref, src_feats_ref,
               ref_pts_ref, src_pts_ref,
               out_ref, rpts_out_ref, spts_out_ref, rfs_ref, sfs_ref):
    ib = pl.program_id(0)

    @pl.when(ib == 0)
    def _prep():
        iota_r = lax.broadcasted_iota(jnp.int32, (_R, _N_REF), 1).astype(jnp.float32)
        oh_r = (ridx_f_ref[...].astype(jnp.float32) == iota_r).astype(jnp.float32)
        rfs_ref[...] = jnp.dot(oh_r, ref_feats_ref[...],
                               preferred_element_type=jnp.float32)
        rpts_out_ref[...] = jnp.dot(oh_r, ref_pts_ref[...],
                                    preferred_element_type=jnp.float32)
        iota_s = lax.broadcasted_iota(jnp.int32, (_S, _N_SRC), 1).astype(jnp.float32)
        oh_s = (sidx_f_ref[...].astype(jnp.float32) == iota_s).astype(jnp.float32)
        sfs_ref[...] = jnp.dot(oh_s, src_feats_ref[...],
                               preferred_element_type=jnp.float32)
        spts_out_ref[...] = jnp.dot(oh_s, src_pts_ref[...],
                                    preferred_element_type=jnp.float32)

    rows = rfs_ref[pl.ds(ib * _RB, _RB), :]
    out_ref[:, :, :_D] = jnp.broadcast_to(rows[:, None, :], (_RB, _S, _D))
    out_ref[:, :, _D:] = jnp.broadcast_to(sfs_ref[...][None, :, :],
                                          (_RB, _S, _D))


_feat_call = pl.pallas_call(
    _feat_body,
    grid=(_R // _RB,),
    in_specs=[
        pl.BlockSpec((_R, 1), lambda i: (0, 0)),
        pl.BlockSpec((_S, 1), lambda i: (0, 0)),
        pl.BlockSpec((_N_REF, _D), lambda i: (0, 0)),
        pl.BlockSpec((_N_SRC, _D), lambda i: (0, 0)),
        pl.BlockSpec((_N_REF, 3), lambda i: (0, 0)),
        pl.BlockSpec((_N_SRC, 3), lambda i: (0, 0)),
    ],
    out_specs=[
        pl.BlockSpec((_RB, _S, 2 * _D), lambda i: (i, 0, 0)),
        pl.BlockSpec((_R, 3), lambda i: (0, 0)),
        pl.BlockSpec((_S, 3), lambda i: (0, 0)),
    ],
    out_shape=[
        jax.ShapeDtypeStruct((_R, _S, 2 * _D), jnp.float32),
        jax.ShapeDtypeStruct((_R, 3), jnp.float32),
        jax.ShapeDtypeStruct((_S, 3), jnp.float32),
    ],
    scratch_shapes=[
        pltpu.VMEM((_R, _D), jnp.float32),
        pltpu.VMEM((_S, _D), jnp.float32),
    ],
    compiler_params=pltpu.CompilerParams(dimension_semantics=("arbitrary",)),
)


def _sc_body(gt_score, packed, corr_out, icorr_out, score_out,
             idx_v, sidx_v, rows_v, srow_v, cbuf_v, pr_v, pc_v, sem):
    wid = lax.axis_index("s") * _NC + lax.axis_index("c")
    base = wid * _RPW
    lane = lax.iota(jnp.int32, 16)
    ones16 = jnp.full((16,), 1.0, jnp.float32)

    pltpu.sync_copy(packed.at[pl.ds(base, _RPW)], idx_v)
    pltpu.sync_copy(packed.at[pl.ds(_R, _S)], sidx_v)
    row_cp = pltpu.async_copy(gt_score.at[idx_v], rows_v, sem)

    def _corr(which, out_hbm):
        pltpu.sync_copy(packed.at[pl.ds(_R + _S + 2 * which * _C, _C)], pr_v)
        pltpu.sync_copy(packed.at[pl.ds(_R + _S + (2 * which + 1) * _C, _C)], pc_v)
        neg16 = -ones16

        def fill(t, carry):
            for j in range(4):
                u = t * 4 + j
                plsc.store_scatter(cbuf_v, [jnp.full((16,), u // 32, jnp.int32),
                                            (u % 32) * 16 + lane], neg16)
            return carry

        lax.fori_loop(0, _RPW * 8, fill, 0)

        def scat(k, carry):
            for j in range(4):
                off = k * 64 + j * 16
                rv = plsc.load_gather(pr_v, [off + lane])
                cv = plsc.load_gather(pc_v, [off + lane])
                lr = rv - base
                m = (lr >= 0) & (lr < _RPW)
                lrc = jnp.clip(lr, 0, _RPW - 1)
                plsc.store_scatter(cbuf_v, [lrc, cv], ones16, mask=m)
            return carry

        lax.fori_loop(0, _C // 64, scat, 0)
        pltpu.sync_copy(cbuf_v, out_hbm.at[pl.ds(base, _RPW)])

    _corr(0, corr_out)
    _corr(1, icorr_out)

    row_cp.wait()

    def srloop(c, carry):
        cols = plsc.load_gather(sidx_v, [c * 16 + lane])
        for r in range(_RPW):
            rvec = jnp.full((16,), r, jnp.int32)
            vals = plsc.load_gather(rows_v, [rvec, cols])
            plsc.store_scatter(srow_v, [rvec, c * 16 + lane], vals)
        return carry

    lax.fori_loop(0, _S // 16, srloop, 0)
    pltpu.sync_copy(srow_v, score_out.at[pl.ds(base, _RPW)])


@functools.cache
def _make_sc_call():
    return pl.kernel(
        _sc_body,
        out_type=[
            jax.ShapeDtypeStruct((_R, _S), jnp.float32),
            jax.ShapeDtypeStruct((_R, _S), jnp.float32),
            jax.ShapeDtypeStruct((_R, _S), jnp.float32),
        ],
        mesh=plsc.VectorSubcoreMesh(core_axis_name="c", subcore_axis_name="s"),
        scratch_types=[
            pltpu.VMEM((_RPW,), jnp.int32),
            pltpu.VMEM((_S,), jnp.int32),
            pltpu.VMEM((_RPW, _N_SRC), jnp.float32),
            pltpu.VMEM((_RPW, _S), jnp.float32),
            pltpu.VMEM((_RPW, _S), jnp.float32),
            pltpu.VMEM((_C,), jnp.int32),
            pltpu.VMEM((_C,), jnp.int32),
            pltpu.SemaphoreType.DMA,
        ],
        compiler_params=pltpu.CompilerParams(needs_layout_passes=False),
    )


def kernel(ref_points, src_points, ref_feats, src_feats, gt_score,
           ref_sample_indices, src_sample_indices, gt_corr_sampled,
           init_corr_sampled):
    ridx = ref_sample_indices.astype(jnp.int32)
    sidx = src_sample_indices.astype(jnp.int32)
    feat, rpts_s, spts_s = _feat_call(ridx.reshape(_R, 1), sidx.reshape(_S, 1),
                                      ref_feats, src_feats,
                                      ref_points, src_points)
    packed = jnp.concatenate([
        ridx, sidx,
        gt_corr_sampled[:, 0], gt_corr_sampled[:, 1],
        init_corr_sampled[:, 0], init_corr_sampled[:, 1],
    ]).astype(jnp.int32)
    corr, icorr, score_s = _make_sc_call()(gt_score, packed)
    return (rpts_s, spts_s, corr, icorr, score_s, feat)


# speedup vs baseline: 1.0108x; 1.0108x over previous
import functools

import jax
import jax.numpy as jnp
from jax import lax
from jax.experimental import pallas as pl
from jax.experimental.pallas import tpu as pltpu
from jax.experimental.pallas import tpu_sc as plsc

_N_REF, _N_SRC, _R, _S, _D, _C = 4096, 4096, 512, 512, 128, 2048
_NC, _NS = 2, 16
_NW = _NC * _NS
_RPW = _R // _NW
_RB = 16


def _feat_body(ridx_f_ref, sidx_f_ref, ref_feats_ref, src_feats_ref,
               ref_pts_ref, src_pts_ref,
               out_ref, rpts_out_ref, spts_out_ref, rfs_ref, sfs_ref):
    ib = pl.program_id(0)

    @pl.when(ib == 0)
    def _prep():
        iota_r = lax.broadcasted_iota(jnp.int32, (_R, _N_REF), 1).astype(jnp.float32)
        oh_r = (ridx_f_ref[...] == iota_r).astype(jnp.float32)
        rfs_ref[...] = jnp.dot(oh_r, ref_feats_ref[...],
                               preferred_element_type=jnp.float32)
        rpts_out_ref[...] = jnp.dot(oh_r, ref_pts_ref[...],
                                    preferred_element_type=jnp.float32)
        iota_s = lax.broadcasted_iota(jnp.int32, (_S, _N_SRC), 1).astype(jnp.float32)
        oh_s = (sidx_f_ref[...] == iota_s).astype(jnp.float32)
        sfs_ref[...] = jnp.dot(oh_s, src_feats_ref[...],
                               preferred_element_type=jnp.float32)
        spts_out_ref[...] = jnp.dot(oh_s, src_pts_ref[...],
                                    preferred_element_type=jnp.float32)

    rows = rfs_ref[pl.ds(ib * _RB, _RB), :]
    out_ref[:, :, :_D] = jnp.broadcast_to(rows[:, None, :], (_RB, _S, _D))
    out_ref[:, :, _D:] = jnp.broadcast_to(sfs_ref[...][None, :, :],
                                          (_RB, _S, _D))


_feat_call = pl.pallas_call(
    _feat_body,
    grid=(_R // _RB,),
    in_specs=[
        pl.BlockSpec((_R, 1), lambda i: (0, 0)),
        pl.BlockSpec((_S, 1), lambda i: (0, 0)),
        pl.BlockSpec((_N_REF, _D), lambda i: (0, 0)),
        pl.BlockSpec((_N_SRC, _D), lambda i: (0, 0)),
        pl.BlockSpec((_N_REF, 3), lambda i: (0, 0)),
        pl.BlockSpec((_N_SRC, 3), lambda i: (0, 0)),
    ],
    out_specs=[
        pl.BlockSpec((_RB, _S, 2 * _D), lambda i: (i, 0, 0)),
        pl.BlockSpec((_R, 3), lambda i: (0, 0)),
        pl.BlockSpec((_S, 3), lambda i: (0, 0)),
    ],
    out_shape=[
        jax.ShapeDtypeStruct((_R, _S, 2 * _D), jnp.float32),
        jax.ShapeDtypeStruct((_R, 3), jnp.float32),
        jax.ShapeDtypeStruct((_S, 3), jnp.float32),
    ],
    scratch_shapes=[
        pltpu.VMEM((_R, _D), jnp.float32),
        pltpu.VMEM((_S, _D), jnp.float32),
    ],
    compiler_params=pltpu.CompilerParams(dimension_semantics=("arbitrary",)),
)


def _sc_body(gt_score, ridx, sidx, pairs, corr_out, icorr_out, score_out,
             idx_v, sidx_v, rows_v, srow_v, cbuf_v, pr_v, pc_v, sem):
    wid = lax.axis_index("s") * _NC + lax.axis_index("c")
    base = wid * _RPW
    lane = lax.iota(jnp.int32, 16)
    ones16 = jnp.full((16,), 1.0, jnp.float32)

    pltpu.sync_copy(ridx.at[pl.ds(base, _RPW)], idx_v)
    pltpu.sync_copy(sidx, sidx_v)
    row_cp = pltpu.async_copy(gt_score.at[idx_v], rows_v, sem)

    def _corr(which, out_hbm):
        pltpu.sync_copy(pairs.at[2 * which], pr_v)
        pltpu.sync_copy(pairs.at[2 * which + 1], pc_v)
        neg16 = -ones16

        def fill(t, carry):
            for j in range(4):
                u = t * 4 + j
                plsc.store_scatter(cbuf_v, [jnp.full((16,), u // 32, jnp.int32),
                                            (u % 32) * 16 + lane], neg16)
            return carry

        lax.fori_loop(0, _RPW * 8, fill, 0)

        def scat(k, carry):
            for j in range(4):
                off = k * 64 + j * 16
                rv = plsc.load_gather(pr_v, [off + lane])
                cv = plsc.load_gather(pc_v, [off + lane])
                lr = rv - base
                m = (lr >= 0) & (lr < _RPW)
                lrc = jnp.clip(lr, 0, _RPW - 1)
                plsc.store_scatter(cbuf_v, [lrc, cv], ones16, mask=m)
            return carry

        lax.fori_loop(0, _C // 64, scat, 0)
        pltpu.sync_copy(cbuf_v, out_hbm.at[pl.ds(base, _RPW)])

    _corr(0, corr_out)
    _corr(1, icorr_out)

    row_cp.wait()

    def srloop(c, carry):
        cols = plsc.load_gather(sidx_v, [c * 16 + lane])
        for r in range(_RPW):
            rvec = jnp.full((16,), r, jnp.int32)
            vals = plsc.load_gather(rows_v, [rvec, cols])
            plsc.store_scatter(srow_v, [rvec, c * 16 + lane], vals)
        return carry

    lax.fori_loop(0, _S // 16, srloop, 0)
    pltpu.sync_copy(srow_v, score_out.at[pl.ds(base, _RPW)])


@functools.cache
def _make_sc_call():
    return pl.kernel(
        _sc_body,
        out_type=[
            jax.ShapeDtypeStruct((_R, _S), jnp.float32),
            jax.ShapeDtypeStruct((_R, _S), jnp.float32),
            jax.ShapeDtypeStruct((_R, _S), jnp.float32),
        ],
        mesh=plsc.VectorSubcoreMesh(core_axis_name="c", subcore_axis_name="s"),
        scratch_types=[
            pltpu.VMEM((_RPW,), jnp.int32),
            pltpu.VMEM((_S,), jnp.int32),
            pltpu.VMEM((_RPW, _N_SRC), jnp.float32),
            pltpu.VMEM((_RPW, _S), jnp.float32),
            pltpu.VMEM((_RPW, _S), jnp.float32),
            pltpu.VMEM((_C,), jnp.int32),
            pltpu.VMEM((_C,), jnp.int32),
            pltpu.SemaphoreType.DMA,
        ],
        compiler_params=pltpu.CompilerParams(needs_layout_passes=False),
    )


def kernel(ref_points, src_points, ref_feats, src_feats, gt_score,
           ref_sample_indices, src_sample_indices, gt_corr_sampled,
           init_corr_sampled):
    ridx = ref_sample_indices.astype(jnp.int32)
    sidx = src_sample_indices.astype(jnp.int32)
    ridx_f = ridx.astype(jnp.float32).reshape(_R, 1)
    sidx_f = sidx.astype(jnp.float32).reshape(_S, 1)
    feat, rpts_s, spts_s = _feat_call(ridx_f, sidx_f, ref_feats, src_feats,
                                      ref_points, src_points)
    pairs = jnp.stack([
        gt_corr_sampled[:, 0], gt_corr_sampled[:, 1],
        init_corr_sampled[:, 0], init_corr_sampled[:, 1],
    ]).astype(jnp.int32)
    corr, icorr, score_s = _make_sc_call()(gt_score, ridx, sidx, pairs)
    return (rpts_s, spts_s, corr, icorr, score_s, feat)
